# baseline (speedup 1.0000x reference)
import functools

import jax
import jax.numpy as jnp
from jax import lax
from jax.experimental import pallas as pl
from jax.experimental.pallas import tpu as pltpu
from jax.experimental.pallas import tpu_sc as plsc

N = 10000
E = 320000
R = 16
NB_BASES = 8
D = 128
L16 = 16

NC, NS = 2, 16
NW = NC * NS
C = 128
WCHUNKS = (-(-E // (C * NW)) + 7) // 8 * 8
NCHUNKS = WCHUNKS * NW
E_PAD = NCHUNKS * C
N_ACC = 10240
ROWS_PER_TILE = N_ACC // NS

@functools.lru_cache(maxsize=None)
def _sc_mesh():
    return plsc.VectorSubcoreMesh(
        core_axis_name="c", subcore_axis_name="s", num_cores=NC, num_subcores=NS)


BN = 1000
NBLK = N // BN


BN_HR = 5000
NB_HR = N // BN_HR


def _hr_body(comb_ref, bases_ref, h_ref, out_ref, w_scr):
    r = pl.program_id(0)
    nb = pl.program_id(1)

    @pl.when(nb == 0)
    def _():
        w = comb_ref[r, 0] * bases_ref[0]
        for b in range(1, NB_BASES):
            w += comb_ref[r, b] * bases_ref[b]
        w_scr[...] = w.astype(jnp.bfloat16)

    out_ref[...] = jnp.dot(h_ref[...].astype(jnp.bfloat16), w_scr[...],
                           preferred_element_type=jnp.float32)


_hr_call = pl.pallas_call(
    _hr_body,
    grid=(R, NB_HR),
    in_specs=[
        pl.BlockSpec(memory_space=pltpu.SMEM),
        pl.BlockSpec((NB_BASES, D, D), lambda r, nb: (0, 0, 0)),
        pl.BlockSpec((BN_HR, D), lambda r, nb: (nb, 0)),
    ],
    out_specs=pl.BlockSpec((BN_HR, D), lambda r, nb: (r * NB_HR + nb, 0)),
    out_shape=jax.ShapeDtypeStruct((R * N, D), jnp.float32),
    scratch_shapes=[pltpu.VMEM((D, D), jnp.bfloat16)],
)


GRP = 16
NGRP = WCHUNKS // GRP
ZR = 64
HALF = WCHUNKS // 2


@functools.lru_cache(maxsize=None)
def _make_edge_kernel():
    scratch = [
        pltpu.VMEM((HALF, C), jnp.int32),
        pltpu.VMEM((HALF, C), jnp.int32),
        pltpu.VMEM((C, D), jnp.float32),
        pltpu.VMEM((C, D), jnp.float32),
        pltpu.VMEM_SHARED((N_ACC, D), jnp.float32),
        pltpu.SemaphoreType.DMA,
        pltpu.SemaphoreType.DMA,
        pltpu.SemaphoreType.DMA,
        pltpu.SemaphoreType.DMA,
    ]

    def body(gidxr, dstr, hr, part_o,
             gidx_v, dst_v, rows0, rows1, acc_sh, sem0, sem1, ssem0, ssem1):
        c = lax.axis_index("c")
        s = lax.axis_index("s")
        wid = c * NS + s

        def _zb(i, carry):
            for j in range(D // L16):
                rows0[i, pl.ds(j * L16, L16)] = jnp.zeros((L16,), jnp.float32)
            return carry
        lax.fori_loop(0, C, _zb, 0)
        base_row = s * ROWS_PER_TILE
        for k in range(ROWS_PER_TILE // C):
            pltpu.sync_copy(rows0, acc_sh.at[pl.ds(base_row + k * C, C)])
        plsc.subcore_barrier()

        def _wait_g(buf, sem):
            pltpu.make_async_copy(hr.at[gidx_v.at[0]], buf, sem).wait()

        def _wait_s(buf, sem):
            pltpu.make_async_copy(buf, acc_sh.at[dst_v.at[0]], sem).wait()

        def _half(h, carry):
            r0 = pl.multiple_of(wid * WCHUNKS + h * HALF, 8)
            pltpu.sync_copy(gidxr.at[pl.ds(r0, HALF)], gidx_v)
            pltpu.sync_copy(dstr.at[pl.ds(r0, HALF)], dst_v)
            pltpu.async_copy(hr.at[gidx_v.at[0]], rows0, sem0)
            pltpu.async_copy(hr.at[gidx_v.at[1]], rows1, sem1)

            def _pair(g, carry2):
                k0 = 2 * g
                _wait_g(rows0, sem0)
                pltpu.async_copy(rows0, acc_sh.at[dst_v.at[k0]], ssem0,
                                 add=True)
                _wait_g(rows1, sem1)
                pltpu.async_copy(rows1, acc_sh.at[dst_v.at[k0 + 1]], ssem1,
                                 add=True)

                @pl.when(g < HALF // 2 - 1)
                def _():
                    _wait_s(rows0, ssem0)
                    pltpu.async_copy(hr.at[gidx_v.at[k0 + 2]], rows0, sem0)
                    _wait_s(rows1, ssem1)
                    pltpu.async_copy(hr.at[gidx_v.at[k0 + 3]], rows1, sem1)
                return carry2
            lax.fori_loop(0, HALF // 2, _pair, 0)
            _wait_s(rows0, ssem0)
            _wait_s(rows1, ssem1)
            return carry
        lax.fori_loop(0, 2, _half, 0)
        plsc.subcore_barrier()

        pltpu.sync_copy(acc_sh.at[pl.ds(base_row, ROWS_PER_TILE)],
                        part_o.at[c, pl.ds(base_row, ROWS_PER_TILE)])

    return pl.kernel(body,
                     out_type=jax.ShapeDtypeStruct((NC, N_ACC, D), jnp.float32),
                     mesh=_sc_mesh(), scratch_types=scratch)


def _gidx_body(src_ref, typ_ref, out_ref):
    out_ref[...] = typ_ref[...] * N + src_ref[...]


_gidx_call = pl.pallas_call(
    _gidx_body,
    grid=(8,),
    in_specs=[pl.BlockSpec((NCHUNKS // 8, C), lambda i: (i, 0)),
              pl.BlockSpec((NCHUNKS // 8, C), lambda i: (i, 0))],
    out_specs=pl.BlockSpec((NCHUNKS // 8, C), lambda i: (i, 0)),
    out_shape=jax.ShapeDtypeStruct((NCHUNKS, C), jnp.int32),
)


@functools.lru_cache(maxsize=None)
def _make_cnt_kernel():
    scratch = [
        pltpu.VMEM((GRP, C), jnp.int32),
        pltpu.VMEM((C, D), jnp.float32),
        pltpu.VMEM((ZR, D), jnp.float32),
        pltpu.VMEM_SHARED((N_ACC, D), jnp.float32),
    ]

    def body(dstr, cnt_o, dst_v, ones_v, zero16_v, cnt_sh):
        c = lax.axis_index("c")
        s = lax.axis_index("s")
        wid = c * NS + s

        def _ob(i, carry):
            for j in range(D // L16):
                sl = pl.ds(j * L16, L16)
                ones_v[i % C, sl] = jnp.ones((L16,), jnp.float32)
                zero16_v[i % ZR, sl] = jnp.zeros((L16,), jnp.float32)
            return carry
        lax.fori_loop(0, C, _ob, 0)

        base_row = s * ROWS_PER_TILE
        for k in range(ROWS_PER_TILE // ZR):
            pltpu.sync_copy(zero16_v, cnt_sh.at[pl.ds(base_row + k * ZR, ZR)])
        plsc.subcore_barrier()

        def _grp(g, carry):
            r0 = pl.multiple_of(wid * WCHUNKS + g * GRP, GRP)
            pltpu.sync_copy(dstr.at[pl.ds(r0, GRP)], dst_v)

            def _eb(k, carry2):
                pltpu.sync_copy(ones_v, cnt_sh.at[dst_v.at[k]], add=True)
                return carry2
            lax.fori_loop(0, GRP, _eb, 0)
            return carry
        lax.fori_loop(0, NGRP, _grp, 0)
        plsc.subcore_barrier()

        pltpu.sync_copy(cnt_sh.at[pl.ds(base_row, ROWS_PER_TILE)],
                        cnt_o.at[c, pl.ds(base_row, ROWS_PER_TILE)])

    return pl.kernel(
        body,
        out_type=jax.ShapeDtypeStruct((NC, N_ACC, D), jnp.float32),
        mesh=_sc_mesh(), scratch_types=scratch)


def _upd_body(cnt_ref, p_ref, h_ref, ws_ref, out_ref):
    deg = cnt_ref[0, :, 0:1] + cnt_ref[1, :, 0:1]
    inv = 1.0 / jnp.maximum(deg, 1.0)
    agg = (p_ref[0] + p_ref[1]) * inv
    out_ref[...] = jnp.maximum(
        agg + jnp.dot(h_ref[...], ws_ref[...],
                      preferred_element_type=jnp.float32), 0.0)


_upd_call = pl.pallas_call(
    _upd_body,
    grid=(NBLK,),
    in_specs=[
        pl.BlockSpec((NC, BN, D), lambda nb: (0, nb, 0)),
        pl.BlockSpec((NC, BN, D), lambda nb: (0, nb, 0)),
        pl.BlockSpec((BN, D), lambda nb: (nb, 0)),
        pl.BlockSpec((D, D), lambda nb: (0, 0)),
    ],
    out_specs=pl.BlockSpec((BN, D), lambda nb: (nb, 0)),
    out_shape=jax.ShapeDtypeStruct((N, D), jnp.float32),
)


NG = 4096
GPW = NG // NW


def _gather_body(h2, idx_hbm, out_hbm, idx_v, rows_v, sem):
    c = lax.axis_index("c")
    s = lax.axis_index("s")
    wid = c * NS + s
    base = pl.multiple_of(wid * GPW, GPW)
    pltpu.sync_copy(idx_hbm.at[pl.ds(base, GPW)], idx_v)
    pltpu.async_copy(h2.at[idx_v], rows_v, sem).wait()
    pltpu.sync_copy(rows_v, out_hbm.at[pl.ds(base, GPW)])


@functools.lru_cache(maxsize=None)
def _make_gather_kernel():
    return pl.kernel(
        _gather_body,
        out_type=jax.ShapeDtypeStruct((NG, D), jnp.float32),
        mesh=_sc_mesh(),
        scratch_types=[
            pltpu.VMEM((GPW,), jnp.int32),
            pltpu.VMEM((GPW, D), jnp.float32),
            pltpu.SemaphoreType.DMA,
        ],
    )


def kernel(x, bases, comb, w_self, edge_index, edge_type, indices):
    src, dst = edge_index[0], edge_index[1]
    padn = E_PAD - E
    ar = jnp.arange(padn, dtype=jnp.int32)
    src_p = jnp.concatenate([src, (ar * 7919) % N]).reshape(NCHUNKS, C)
    typ_p = jnp.concatenate([edge_type, ar % R]).reshape(NCHUNKS, C)
    dst_p = jnp.concatenate([dst, N + ar % (N_ACC - N)]).reshape(NCHUNKS, C)

    h = x
    gidx_p = _gidx_call(src_p, typ_p)
    cnt = _make_cnt_kernel()(dst_p)
    for layer in range(bases.shape[0]):
        hr = _hr_call(comb[layer], bases[layer], h)
        part = _make_edge_kernel()(gidx_p, dst_p, hr)
        h = _upd_call(cnt, part, h, w_self[layer])
    return _make_gather_kernel()(h, indices)


# speedup vs baseline: 1.2182x; 1.2182x over previous
import functools

import jax
import jax.numpy as jnp
from jax import lax
from jax.experimental import pallas as pl
from jax.experimental.pallas import tpu as pltpu
from jax.experimental.pallas import tpu_sc as plsc

N = 10000
E = 320000
R = 16
NB_BASES = 8
D = 128
L16 = 16

NC, NS = 2, 16
NW = NC * NS
C = 128
WCHUNKS = (-(-E // (C * NW)) + 7) // 8 * 8
NCHUNKS = WCHUNKS * NW
E_PAD = NCHUNKS * C
N_ACC = 10240
ROWS_PER_TILE = N_ACC // NS

@functools.lru_cache(maxsize=None)
def _sc_mesh():
    return plsc.VectorSubcoreMesh(
        core_axis_name="c", subcore_axis_name="s", num_cores=NC, num_subcores=NS)


BN = 1000
NBLK = N // BN


BN_HR = 5000
NB_HR = N // BN_HR


def _hr_body(comb_ref, bases_ref, h_ref, out_ref, w_scr):
    r = pl.program_id(0)
    nb = pl.program_id(1)

    @pl.when(nb == 0)
    def _():
        w = comb_ref[r, 0] * bases_ref[0]
        for b in range(1, NB_BASES):
            w += comb_ref[r, b] * bases_ref[b]
        w_scr[...] = w.astype(jnp.bfloat16)

    out_ref[...] = jnp.dot(h_ref[...].astype(jnp.bfloat16), w_scr[...],
                           preferred_element_type=jnp.float32)


_hr_call = pl.pallas_call(
    _hr_body,
    grid=(R, NB_HR),
    in_specs=[
        pl.BlockSpec(memory_space=pltpu.SMEM),
        pl.BlockSpec((NB_BASES, D, D), lambda r, nb: (0, 0, 0)),
        pl.BlockSpec((BN_HR, D), lambda r, nb: (nb, 0)),
    ],
    out_specs=pl.BlockSpec((BN_HR, D), lambda r, nb: (r * NB_HR + nb, 0)),
    out_shape=jax.ShapeDtypeStruct((R * N, D), jnp.float32),
    scratch_shapes=[pltpu.VMEM((D, D), jnp.bfloat16)],
)


GRP = 16
NGRP = WCHUNKS // GRP
ZR = 64
HALF = WCHUNKS // 2


@functools.lru_cache(maxsize=None)
def _make_edge_kernel():
    scratch = [
        pltpu.VMEM((HALF, C), jnp.int32),
        pltpu.VMEM((HALF, C), jnp.int32),
        pltpu.VMEM((C, D), jnp.float32),
        pltpu.VMEM((C, D), jnp.float32),
        pltpu.VMEM_SHARED((N_ACC, D), jnp.float32),
        pltpu.SemaphoreType.DMA,
        pltpu.SemaphoreType.DMA,
    ]

    def body(gidxr, dstr, hr, part_o,
             gidx_v, dst_v, rows0, rows1, acc_sh, sem0, sem1):
        c = lax.axis_index("c")
        s = lax.axis_index("s")
        wid = c * NS + s

        def _zb(i, carry):
            for j in range(D // L16):
                rows0[i, pl.ds(j * L16, L16)] = jnp.zeros((L16,), jnp.float32)
            return carry
        lax.fori_loop(0, C, _zb, 0)
        base_row = s * ROWS_PER_TILE
        for k in range(ROWS_PER_TILE // C):
            pltpu.sync_copy(rows0, acc_sh.at[pl.ds(base_row + k * C, C)])
        plsc.subcore_barrier()

        def _half(h, carry):
            r0 = pl.multiple_of(wid * WCHUNKS + h * HALF, 8)
            pltpu.sync_copy(gidxr.at[pl.ds(r0, HALF)], gidx_v)
            pltpu.sync_copy(dstr.at[pl.ds(r0, HALF)], dst_v)
            pltpu.async_copy(hr.at[gidx_v.at[0]], rows0, sem0)

            def _pair(g, carry2):
                k0 = 2 * g
                pltpu.async_copy(hr.at[gidx_v.at[k0 + 1]], rows1, sem1)
                pltpu.make_async_copy(hr.at[gidx_v.at[k0]], rows0, sem0).wait()
                pltpu.sync_copy(rows0, acc_sh.at[dst_v.at[k0]], add=True)

                @pl.when(g < HALF // 2 - 1)
                def _():
                    pltpu.async_copy(hr.at[gidx_v.at[k0 + 2]], rows0, sem0)
                pltpu.make_async_copy(hr.at[gidx_v.at[k0 + 1]], rows1,
                                      sem1).wait()
                pltpu.sync_copy(rows1, acc_sh.at[dst_v.at[k0 + 1]], add=True)
                return carry2
            lax.fori_loop(0, HALF // 2, _pair, 0)
            return carry
        lax.fori_loop(0, 2, _half, 0)
        plsc.subcore_barrier()

        pltpu.sync_copy(acc_sh.at[pl.ds(base_row, ROWS_PER_TILE)],
                        part_o.at[c, pl.ds(base_row, ROWS_PER_TILE)])

    return pl.kernel(body,
                     out_type=jax.ShapeDtypeStruct((NC, N_ACC, D), jnp.float32),
                     mesh=_sc_mesh(), scratch_types=scratch)


def _upd_hr_body(comb_ref, bases_ref, cnt_ref, p_ref, h_ref, ws_ref,
                 hr_ref, hnew_ref, h1_scr):
    r = pl.program_id(1)

    @pl.when(r == 0)
    def _():
        deg = cnt_ref[0, :, 0:1] + cnt_ref[1, :, 0:1]
        inv = 1.0 / jnp.maximum(deg, 1.0)
        agg = (p_ref[0] + p_ref[1]) * inv
        h1 = jnp.maximum(
            agg + jnp.dot(h_ref[...], ws_ref[...],
                          preferred_element_type=jnp.float32), 0.0)
        hnew_ref[...] = h1
        h1_scr[...] = h1.astype(jnp.bfloat16)

    w = comb_ref[r, 0] * bases_ref[0]
    for b in range(1, NB_BASES):
        w += comb_ref[r, b] * bases_ref[b]
    hr_ref[...] = jnp.dot(h1_scr[...], w.astype(jnp.bfloat16),
                          preferred_element_type=jnp.float32)


_upd_hr_call = pl.pallas_call(
    _upd_hr_body,
    grid=(NB_HR, R),
    in_specs=[
        pl.BlockSpec(memory_space=pltpu.SMEM),
        pl.BlockSpec((NB_BASES, D, D), lambda nb, r: (0, 0, 0)),
        pl.BlockSpec((NC, BN_HR, D), lambda nb, r: (0, nb, 0)),
        pl.BlockSpec((NC, BN_HR, D), lambda nb, r: (0, nb, 0)),
        pl.BlockSpec((BN_HR, D), lambda nb, r: (nb, 0)),
        pl.BlockSpec((D, D), lambda nb, r: (0, 0)),
    ],
    out_specs=[
        pl.BlockSpec((BN_HR, D), lambda nb, r: (r * NB_HR + nb, 0)),
        pl.BlockSpec((BN_HR, D), lambda nb, r: (nb, 0)),
    ],
    out_shape=[jax.ShapeDtypeStruct((R * N, D), jnp.float32),
               jax.ShapeDtypeStruct((N, D), jnp.float32)],
    scratch_shapes=[pltpu.VMEM((BN_HR, D), jnp.bfloat16)],
)


@functools.lru_cache(maxsize=None)
def _make_cnt_kernel():
    scratch = [
        pltpu.VMEM((GRP, C), jnp.int32),
        pltpu.VMEM((GRP, C), jnp.int32),
        pltpu.VMEM((GRP, C), jnp.int32),
        pltpu.VMEM((GRP, C), jnp.int32),
        pltpu.VMEM((C, D), jnp.float32),
        pltpu.VMEM((ZR, D), jnp.float32),
        pltpu.VMEM_SHARED((N_ACC, D), jnp.float32),
    ]

    def body(srcr, typr, dstr, cnt_o, gidx_o,
             dst_v, src_v, typ_v, gidx_v, ones_v, zero16_v, cnt_sh):
        c = lax.axis_index("c")
        s = lax.axis_index("s")
        wid = c * NS + s

        def _ob(i, carry):
            for j in range(D // L16):
                sl = pl.ds(j * L16, L16)
                ones_v[i % C, sl] = jnp.ones((L16,), jnp.float32)
                zero16_v[i % ZR, sl] = jnp.zeros((L16,), jnp.float32)
            return carry
        lax.fori_loop(0, C, _ob, 0)

        base_row = s * ROWS_PER_TILE
        for k in range(ROWS_PER_TILE // ZR):
            pltpu.sync_copy(zero16_v, cnt_sh.at[pl.ds(base_row + k * ZR, ZR)])
        plsc.subcore_barrier()

        def _grp(g, carry):
            r0 = pl.multiple_of(wid * WCHUNKS + g * GRP, GRP)
            pltpu.sync_copy(dstr.at[pl.ds(r0, GRP)], dst_v)
            pltpu.sync_copy(srcr.at[pl.ds(r0, GRP)], src_v)
            pltpu.sync_copy(typr.at[pl.ds(r0, GRP)], typ_v)

            def _eb(k, carry2):
                for j in range(C // L16):
                    sl = pl.ds(j * L16, L16)
                    gidx_v[k, sl] = typ_v[k, sl] * N + src_v[k, sl]
                pltpu.sync_copy(ones_v, cnt_sh.at[dst_v.at[k]], add=True)
                return carry2
            lax.fori_loop(0, GRP, _eb, 0)
            pltpu.sync_copy(gidx_v, gidx_o.at[pl.ds(r0, GRP)])
            return carry
        lax.fori_loop(0, NGRP, _grp, 0)
        plsc.subcore_barrier()

        pltpu.sync_copy(cnt_sh.at[pl.ds(base_row, ROWS_PER_TILE)],
                        cnt_o.at[c, pl.ds(base_row, ROWS_PER_TILE)])

    return pl.kernel(
        body,
        out_type=(jax.ShapeDtypeStruct((NC, N_ACC, D), jnp.float32),
                  jax.ShapeDtypeStruct((NCHUNKS, C), jnp.int32)),
        mesh=_sc_mesh(), scratch_types=scratch)


def _upd_body(cnt_ref, p_ref, h_ref, ws_ref, out_ref):
    deg = cnt_ref[0, :, 0:1] + cnt_ref[1, :, 0:1]
    inv = 1.0 / jnp.maximum(deg, 1.0)
    agg = (p_ref[0] + p_ref[1]) * inv
    out_ref[...] = jnp.maximum(
        agg + jnp.dot(h_ref[...], ws_ref[...],
                      preferred_element_type=jnp.float32), 0.0)


_upd_call = pl.pallas_call(
    _upd_body,
    grid=(NBLK,),
    in_specs=[
        pl.BlockSpec((NC, BN, D), lambda nb: (0, nb, 0)),
        pl.BlockSpec((NC, BN, D), lambda nb: (0, nb, 0)),
        pl.BlockSpec((BN, D), lambda nb: (nb, 0)),
        pl.BlockSpec((D, D), lambda nb: (0, 0)),
    ],
    out_specs=pl.BlockSpec((BN, D), lambda nb: (nb, 0)),
    out_shape=jax.ShapeDtypeStruct((N, D), jnp.float32),
)


NG = 4096
GPW = NG // NW


def _gather_body(h2, idx_hbm, out_hbm, idx_v, rows_v, sem):
    c = lax.axis_index("c")
    s = lax.axis_index("s")
    wid = c * NS + s
    base = pl.multiple_of(wid * GPW, GPW)
    pltpu.sync_copy(idx_hbm.at[pl.ds(base, GPW)], idx_v)
    pltpu.async_copy(h2.at[idx_v], rows_v, sem).wait()
    pltpu.sync_copy(rows_v, out_hbm.at[pl.ds(base, GPW)])


@functools.lru_cache(maxsize=None)
def _make_gather_kernel():
    return pl.kernel(
        _gather_body,
        out_type=jax.ShapeDtypeStruct((NG, D), jnp.float32),
        mesh=_sc_mesh(),
        scratch_types=[
            pltpu.VMEM((GPW,), jnp.int32),
            pltpu.VMEM((GPW, D), jnp.float32),
            pltpu.SemaphoreType.DMA,
        ],
    )


def kernel(x, bases, comb, w_self, edge_index, edge_type, indices):
    src, dst = edge_index[0], edge_index[1]
    padn = E_PAD - E
    ar = jnp.arange(padn, dtype=jnp.int32)
    src_p = jnp.concatenate([src, (ar * 7919) % N]).reshape(NCHUNKS, C)
    typ_p = jnp.concatenate([edge_type, ar % R]).reshape(NCHUNKS, C)
    dst_p = jnp.concatenate([dst, N + ar % (N_ACC - N)]).reshape(NCHUNKS, C)

    cnt, gidx_p = _make_cnt_kernel()(src_p, typ_p, dst_p)
    hr = _hr_call(comb[0], bases[0], x)
    part = _make_edge_kernel()(gidx_p, dst_p, hr)
    hr2, h1 = _upd_hr_call(comb[1], bases[1], cnt, part, x, w_self[0])
    part2 = _make_edge_kernel()(gidx_p, dst_p, hr2)
    h2 = _upd_call(cnt, part2, h1, w_self[1])
    return _make_gather_kernel()(h2, indices)


# speedup vs baseline: 1.2530x; 1.0286x over previous
import functools

import jax
import jax.numpy as jnp
from jax import lax
from jax.experimental import pallas as pl
from jax.experimental.pallas import tpu as pltpu
from jax.experimental.pallas import tpu_sc as plsc

N = 10000
E = 320000
R = 16
NB_BASES = 8
D = 128
L16 = 16

NC, NS = 2, 16
NW = NC * NS
C = 128
WCHUNKS = (-(-E // (C * NW)) + 7) // 8 * 8
NCHUNKS = WCHUNKS * NW
E_PAD = NCHUNKS * C
N_ACC = 10240
ROWS_PER_TILE = N_ACC // NS

@functools.lru_cache(maxsize=None)
def _sc_mesh():
    return plsc.VectorSubcoreMesh(
        core_axis_name="c", subcore_axis_name="s", num_cores=NC, num_subcores=NS)


BN = 1000
NBLK = N // BN


BN_HR = 5000
NB_HR = N // BN_HR


def _hr_body(comb_ref, bases_ref, h_ref, out_ref, h_scr):
    r = pl.program_id(1)

    @pl.when(r == 0)
    def _():
        h_scr[...] = h_ref[...].astype(jnp.bfloat16)

    w = comb_ref[r, 0] * bases_ref[0]
    for b in range(1, NB_BASES):
        w += comb_ref[r, b] * bases_ref[b]
    out_ref[...] = jnp.dot(h_scr[...], w.astype(jnp.bfloat16),
                           preferred_element_type=jnp.float32)


_hr_call = pl.pallas_call(
    _hr_body,
    grid=(NB_HR, R),
    in_specs=[
        pl.BlockSpec(memory_space=pltpu.SMEM),
        pl.BlockSpec((NB_BASES, D, D), lambda nb, r: (0, 0, 0)),
        pl.BlockSpec((BN_HR, D), lambda nb, r: (nb, 0)),
    ],
    out_specs=pl.BlockSpec((BN_HR, D), lambda nb, r: (r * NB_HR + nb, 0)),
    out_shape=jax.ShapeDtypeStruct((R * N, D), jnp.float32),
    scratch_shapes=[pltpu.VMEM((BN_HR, D), jnp.bfloat16)],
)


GRP = 16
NGRP = WCHUNKS // GRP
ZR = 64
HALF = WCHUNKS // 2


@functools.lru_cache(maxsize=None)
def _make_edge_kernel():
    scratch = [
        pltpu.VMEM((HALF, C), jnp.int32),
        pltpu.VMEM((HALF, C), jnp.int32),
        pltpu.VMEM((C, D), jnp.float32),
        pltpu.VMEM((C, D), jnp.float32),
        pltpu.VMEM_SHARED((N_ACC, D), jnp.float32),
        pltpu.SemaphoreType.DMA,
        pltpu.SemaphoreType.DMA,
    ]

    def body(gidxr, dstr, hr, part_o,
             gidx_v, dst_v, rows0, rows1, acc_sh, sem0, sem1):
        c = lax.axis_index("c")
        s = lax.axis_index("s")
        wid = c * NS + s

        def _zb(i, carry):
            for j in range(D // L16):
                rows0[i, pl.ds(j * L16, L16)] = jnp.zeros((L16,), jnp.float32)
            return carry
        lax.fori_loop(0, C, _zb, 0)
        base_row = s * ROWS_PER_TILE
        for k in range(ROWS_PER_TILE // C):
            pltpu.sync_copy(rows0, acc_sh.at[pl.ds(base_row + k * C, C)])
        plsc.subcore_barrier()

        def _half(h, carry):
            r0 = pl.multiple_of(wid * WCHUNKS + h * HALF, 8)
            pltpu.sync_copy(gidxr.at[pl.ds(r0, HALF)], gidx_v)
            pltpu.sync_copy(dstr.at[pl.ds(r0, HALF)], dst_v)
            pltpu.async_copy(hr.at[gidx_v.at[0]], rows0, sem0)

            def _pair(g, carry2):
                k0 = 2 * g
                pltpu.async_copy(hr.at[gidx_v.at[k0 + 1]], rows1, sem1)
                pltpu.make_async_copy(hr.at[gidx_v.at[k0]], rows0, sem0).wait()
                pltpu.sync_copy(rows0, acc_sh.at[dst_v.at[k0]], add=True)

                @pl.when(g < HALF // 2 - 1)
                def _():
                    pltpu.async_copy(hr.at[gidx_v.at[k0 + 2]], rows0, sem0)
                pltpu.make_async_copy(hr.at[gidx_v.at[k0 + 1]], rows1,
                                      sem1).wait()
                pltpu.sync_copy(rows1, acc_sh.at[dst_v.at[k0 + 1]], add=True)
                return carry2
            lax.fori_loop(0, HALF // 2, _pair, 0)
            return carry
        lax.fori_loop(0, 2, _half, 0)
        plsc.subcore_barrier()

        pltpu.sync_copy(acc_sh.at[pl.ds(base_row, ROWS_PER_TILE)],
                        part_o.at[c, pl.ds(base_row, ROWS_PER_TILE)])

    return pl.kernel(body,
                     out_type=jax.ShapeDtypeStruct((NC, N_ACC, D), jnp.float32),
                     mesh=_sc_mesh(), scratch_types=scratch)


def _upd_hr_body(comb_ref, bases_ref, cnt_ref, p_ref, h_ref, ws_ref,
                 hr_ref, hnew_ref, h1_scr):
    r = pl.program_id(1)

    @pl.when(r == 0)
    def _():
        deg = cnt_ref[0, :, 0:1] + cnt_ref[1, :, 0:1]
        inv = 1.0 / jnp.maximum(deg, 1.0)
        agg = (p_ref[0] + p_ref[1]) * inv
        h1 = jnp.maximum(
            agg + jnp.dot(h_ref[...], ws_ref[...],
                          preferred_element_type=jnp.float32), 0.0)
        hnew_ref[...] = h1
        h1_scr[...] = h1.astype(jnp.bfloat16)

    w = comb_ref[r, 0] * bases_ref[0]
    for b in range(1, NB_BASES):
        w += comb_ref[r, b] * bases_ref[b]
    hr_ref[...] = jnp.dot(h1_scr[...], w.astype(jnp.bfloat16),
                          preferred_element_type=jnp.float32)


_upd_hr_call = pl.pallas_call(
    _upd_hr_body,
    grid=(NB_HR, R),
    in_specs=[
        pl.BlockSpec(memory_space=pltpu.SMEM),
        pl.BlockSpec((NB_BASES, D, D), lambda nb, r: (0, 0, 0)),
        pl.BlockSpec((NC, BN_HR, D), lambda nb, r: (0, nb, 0)),
        pl.BlockSpec((NC, BN_HR, D), lambda nb, r: (0, nb, 0)),
        pl.BlockSpec((BN_HR, D), lambda nb, r: (nb, 0)),
        pl.BlockSpec((D, D), lambda nb, r: (0, 0)),
    ],
    out_specs=[
        pl.BlockSpec((BN_HR, D), lambda nb, r: (r * NB_HR + nb, 0)),
        pl.BlockSpec((BN_HR, D), lambda nb, r: (nb, 0)),
    ],
    out_shape=[jax.ShapeDtypeStruct((R * N, D), jnp.float32),
               jax.ShapeDtypeStruct((N, D), jnp.float32)],
    scratch_shapes=[pltpu.VMEM((BN_HR, D), jnp.bfloat16)],
)


@functools.lru_cache(maxsize=None)
def _make_cnt_kernel():
    scratch = [
        pltpu.VMEM((GRP, C), jnp.int32),
        pltpu.VMEM((C, D), jnp.float32),
        pltpu.VMEM((ZR, D), jnp.float32),
        pltpu.VMEM_SHARED((N_ACC, D), jnp.float32),
    ]

    def body(dstr, cnt_o, dst_v, ones_v, zero16_v, cnt_sh):
        c = lax.axis_index("c")
        s = lax.axis_index("s")
        wid = c * NS + s

        def _ob(i, carry):
            for j in range(D // L16):
                sl = pl.ds(j * L16, L16)
                ones_v[i % C, sl] = jnp.ones((L16,), jnp.float32)
                zero16_v[i % ZR, sl] = jnp.zeros((L16,), jnp.float32)
            return carry
        lax.fori_loop(0, C, _ob, 0)

        base_row = s * ROWS_PER_TILE
        for k in range(ROWS_PER_TILE // ZR):
            pltpu.sync_copy(zero16_v, cnt_sh.at[pl.ds(base_row + k * ZR, ZR)])
        plsc.subcore_barrier()

        def _grp(g, carry):
            r0 = pl.multiple_of(wid * WCHUNKS + g * GRP, GRP)
            pltpu.sync_copy(dstr.at[pl.ds(r0, GRP)], dst_v)

            def _eb(k, carry2):
                pltpu.sync_copy(ones_v, cnt_sh.at[dst_v.at[k]], add=True)
                return carry2
            lax.fori_loop(0, GRP, _eb, 0)
            return carry
        lax.fori_loop(0, NGRP, _grp, 0)
        plsc.subcore_barrier()

        pltpu.sync_copy(cnt_sh.at[pl.ds(base_row, ROWS_PER_TILE)],
                        cnt_o.at[c, pl.ds(base_row, ROWS_PER_TILE)])

    return pl.kernel(
        body,
        out_type=jax.ShapeDtypeStruct((NC, N_ACC, D), jnp.float32),
        mesh=_sc_mesh(), scratch_types=scratch)


def _gidx_body(src_ref, typ_ref, out_ref):
    out_ref[...] = typ_ref[...] * N + src_ref[...]


_gidx_call = pl.pallas_call(
    _gidx_body,
    grid=(4,),
    in_specs=[pl.BlockSpec((NCHUNKS // 4, C), lambda i: (i, 0)),
              pl.BlockSpec((NCHUNKS // 4, C), lambda i: (i, 0))],
    out_specs=pl.BlockSpec((NCHUNKS // 4, C), lambda i: (i, 0)),
    out_shape=jax.ShapeDtypeStruct((NCHUNKS, C), jnp.int32),
)


def _upd_body(cnt_ref, p_ref, h_ref, ws_ref, out_ref):
    deg = cnt_ref[0, :, 0:1] + cnt_ref[1, :, 0:1]
    inv = 1.0 / jnp.maximum(deg, 1.0)
    agg = (p_ref[0] + p_ref[1]) * inv
    out_ref[...] = jnp.maximum(
        agg + jnp.dot(h_ref[...], ws_ref[...],
                      preferred_element_type=jnp.float32), 0.0)


_upd_call = pl.pallas_call(
    _upd_body,
    grid=(NBLK,),
    in_specs=[
        pl.BlockSpec((NC, BN, D), lambda nb: (0, nb, 0)),
        pl.BlockSpec((NC, BN, D), lambda nb: (0, nb, 0)),
        pl.BlockSpec((BN, D), lambda nb: (nb, 0)),
        pl.BlockSpec((D, D), lambda nb: (0, 0)),
    ],
    out_specs=pl.BlockSpec((BN, D), lambda nb: (nb, 0)),
    out_shape=jax.ShapeDtypeStruct((N, D), jnp.float32),
)


NG = 4096
GPW = NG // NW


def _gather_body(h2, idx_hbm, out_hbm, idx_v, rows_v, sem):
    c = lax.axis_index("c")
    s = lax.axis_index("s")
    wid = c * NS + s
    base = pl.multiple_of(wid * GPW, GPW)
    pltpu.sync_copy(idx_hbm.at[pl.ds(base, GPW)], idx_v)
    pltpu.async_copy(h2.at[idx_v], rows_v, sem).wait()
    pltpu.sync_copy(rows_v, out_hbm.at[pl.ds(base, GPW)])


@functools.lru_cache(maxsize=None)
def _make_gather_kernel():
    return pl.kernel(
        _gather_body,
        out_type=jax.ShapeDtypeStruct((NG, D), jnp.float32),
        mesh=_sc_mesh(),
        scratch_types=[
            pltpu.VMEM((GPW,), jnp.int32),
            pltpu.VMEM((GPW, D), jnp.float32),
            pltpu.SemaphoreType.DMA,
        ],
    )


def kernel(x, bases, comb, w_self, edge_index, edge_type, indices):
    src, dst = edge_index[0], edge_index[1]
    padn = E_PAD - E
    ar = jnp.arange(padn, dtype=jnp.int32)
    src_p = jnp.concatenate([src, (ar * 7919) % N]).reshape(NCHUNKS, C)
    typ_p = jnp.concatenate([edge_type, ar % R]).reshape(NCHUNKS, C)
    dst_p = jnp.concatenate([dst, N + ar % (N_ACC - N)]).reshape(NCHUNKS, C)

    gidx_p = _gidx_call(src_p, typ_p)
    cnt = _make_cnt_kernel()(dst_p)
    hr = _hr_call(comb[0], bases[0], x)
    part = _make_edge_kernel()(gidx_p, dst_p, hr)
    hr2, h1 = _upd_hr_call(comb[1], bases[1], cnt, part, x, w_self[0])
    part2 = _make_edge_kernel()(gidx_p, dst_p, hr2)
    h2 = _upd_call(cnt, part2, h1, w_self[1])
    return _make_gather_kernel()(h2, indices)


# speedup vs baseline: 1.2721x; 1.0152x over previous
import functools

import jax
import jax.numpy as jnp
from jax import lax
from jax.experimental import pallas as pl
from jax.experimental.pallas import tpu as pltpu
from jax.experimental.pallas import tpu_sc as plsc

N = 10000
E = 320000
R = 16
NB_BASES = 8
D = 128
L16 = 16

NC, NS = 2, 16
NW = NC * NS
C = 128
WCHUNKS = (-(-E // (C * NW)) + 7) // 8 * 8
NCHUNKS = WCHUNKS * NW
E_PAD = NCHUNKS * C
N_ACC = 10240
ROWS_PER_TILE = N_ACC // NS

@functools.lru_cache(maxsize=None)
def _sc_mesh():
    return plsc.VectorSubcoreMesh(
        core_axis_name="c", subcore_axis_name="s", num_cores=NC, num_subcores=NS)


BN = 1000
NBLK = N // BN


BN_HR = 5000
NB_HR = N // BN_HR


def _hr_body(comb_ref, bases_ref, h_ref, out_ref, h_scr):
    r = pl.program_id(1)

    @pl.when(r == 0)
    def _():
        h_scr[...] = h_ref[...].astype(jnp.bfloat16)

    w = comb_ref[r, 0] * bases_ref[0]
    for b in range(1, NB_BASES):
        w += comb_ref[r, b] * bases_ref[b]
    out_ref[...] = jnp.dot(h_scr[...], w.astype(jnp.bfloat16),
                           preferred_element_type=jnp.float32)


_hr_call = pl.pallas_call(
    _hr_body,
    grid=(NB_HR, R),
    in_specs=[
        pl.BlockSpec(memory_space=pltpu.SMEM),
        pl.BlockSpec((NB_BASES, D, D), lambda nb, r: (0, 0, 0)),
        pl.BlockSpec((BN_HR, D), lambda nb, r: (nb, 0)),
    ],
    out_specs=pl.BlockSpec((BN_HR, D), lambda nb, r: (r * NB_HR + nb, 0)),
    out_shape=jax.ShapeDtypeStruct((R * N, D), jnp.float32),
    scratch_shapes=[pltpu.VMEM((BN_HR, D), jnp.bfloat16)],
)


GRP = 16
NGRP = WCHUNKS // GRP
ZR = 64
HALF = WCHUNKS // 2


@functools.lru_cache(maxsize=None)
def _make_edge_kernel():
    scratch = [
        pltpu.VMEM((HALF, C), jnp.int32),
        pltpu.VMEM((HALF, C), jnp.int32),
        pltpu.VMEM((C, D), jnp.float32),
        pltpu.VMEM((C, D), jnp.float32),
        pltpu.VMEM_SHARED((N_ACC, D), jnp.float32),
        pltpu.SemaphoreType.DMA,
        pltpu.SemaphoreType.DMA,
    ]

    def body(gidxr, dstr, hr, part_o,
             gidx_v, dst_v, rows0, rows1, acc_sh, sem0, sem1):
        c = lax.axis_index("c")
        s = lax.axis_index("s")
        wid = c * NS + s

        def _zb(i, carry):
            for j in range(D // L16):
                rows0[i, pl.ds(j * L16, L16)] = jnp.zeros((L16,), jnp.float32)
            return carry
        lax.fori_loop(0, C, _zb, 0)
        base_row = s * ROWS_PER_TILE
        for k in range(ROWS_PER_TILE // C):
            pltpu.sync_copy(rows0, acc_sh.at[pl.ds(base_row + k * C, C)])
        plsc.subcore_barrier()

        def _half(h, carry):
            r0 = pl.multiple_of(wid * WCHUNKS + h * HALF, 8)
            pltpu.sync_copy(gidxr.at[pl.ds(r0, HALF)], gidx_v)
            pltpu.sync_copy(dstr.at[pl.ds(r0, HALF)], dst_v)
            pltpu.async_copy(hr.at[gidx_v.at[0]], rows0, sem0)

            def _pair(g, carry2):
                k0 = 2 * g
                pltpu.async_copy(hr.at[gidx_v.at[k0 + 1]], rows1, sem1)
                pltpu.make_async_copy(hr.at[gidx_v.at[k0]], rows0, sem0).wait()
                pltpu.sync_copy(rows0, acc_sh.at[dst_v.at[k0]], add=True)

                @pl.when(g < HALF // 2 - 1)
                def _():
                    pltpu.async_copy(hr.at[gidx_v.at[k0 + 2]], rows0, sem0)
                pltpu.make_async_copy(hr.at[gidx_v.at[k0 + 1]], rows1,
                                      sem1).wait()
                pltpu.sync_copy(rows1, acc_sh.at[dst_v.at[k0 + 1]], add=True)
                return carry2
            lax.fori_loop(0, HALF // 2, _pair, 0)
            return carry
        lax.fori_loop(0, 2, _half, 0)
        plsc.subcore_barrier()

        pltpu.sync_copy(acc_sh.at[pl.ds(base_row, ROWS_PER_TILE)],
                        part_o.at[c, pl.ds(base_row, ROWS_PER_TILE)])

    return pl.kernel(body,
                     out_type=jax.ShapeDtypeStruct((NC, N_ACC, D), jnp.float32),
                     mesh=_sc_mesh(), scratch_types=scratch)


def _upd_hr_body(comb_ref, bases_ref, cnt_ref, p_ref, h_ref, ws_ref, ws2_ref,
                 hr_ref, y1_ref, h1_scr):
    r = pl.program_id(1)

    @pl.when(r == 0)
    def _():
        deg = cnt_ref[0, :, 0:1] + cnt_ref[1, :, 0:1]
        inv = 1.0 / jnp.maximum(deg, 1.0)
        agg = (p_ref[0] + p_ref[1]) * inv
        h1 = jnp.maximum(
            agg + jnp.dot(h_ref[...], ws_ref[...],
                          preferred_element_type=jnp.float32), 0.0)
        h1b = h1.astype(jnp.bfloat16)
        h1_scr[...] = h1b
        y1_ref[...] = jnp.dot(h1b, ws2_ref[...].astype(jnp.bfloat16),
                              preferred_element_type=jnp.float32)

    w = comb_ref[r, 0] * bases_ref[0]
    for b in range(1, NB_BASES):
        w += comb_ref[r, b] * bases_ref[b]
    hr_ref[...] = jnp.dot(h1_scr[...], w.astype(jnp.bfloat16),
                          preferred_element_type=jnp.float32)


_upd_hr_call = pl.pallas_call(
    _upd_hr_body,
    grid=(NB_HR, R),
    in_specs=[
        pl.BlockSpec(memory_space=pltpu.SMEM),
        pl.BlockSpec((NB_BASES, D, D), lambda nb, r: (0, 0, 0)),
        pl.BlockSpec((NC, BN_HR, D), lambda nb, r: (0, nb, 0)),
        pl.BlockSpec((NC, BN_HR, D), lambda nb, r: (0, nb, 0)),
        pl.BlockSpec((BN_HR, D), lambda nb, r: (nb, 0)),
        pl.BlockSpec((D, D), lambda nb, r: (0, 0)),
        pl.BlockSpec((D, D), lambda nb, r: (0, 0)),
    ],
    out_specs=[
        pl.BlockSpec((BN_HR, D), lambda nb, r: (r * NB_HR + nb, 0)),
        pl.BlockSpec((BN_HR, D), lambda nb, r: (nb, 0)),
    ],
    out_shape=[jax.ShapeDtypeStruct((R * N, D), jnp.float32),
               jax.ShapeDtypeStruct((N, D), jnp.float32)],
    scratch_shapes=[pltpu.VMEM((BN_HR, D), jnp.bfloat16)],
)


@functools.lru_cache(maxsize=None)
def _make_cnt_kernel():
    scratch = [
        pltpu.VMEM((GRP, C), jnp.int32),
        pltpu.VMEM((C, D), jnp.float32),
        pltpu.VMEM((ZR, D), jnp.float32),
        pltpu.VMEM_SHARED((N_ACC, D), jnp.float32),
    ]

    def body(dstr, cnt_o, dst_v, ones_v, zero16_v, cnt_sh):
        c = lax.axis_index("c")
        s = lax.axis_index("s")
        wid = c * NS + s

        def _ob(i, carry):
            for j in range(D // L16):
                sl = pl.ds(j * L16, L16)
                ones_v[i % C, sl] = jnp.ones((L16,), jnp.float32)
                zero16_v[i % ZR, sl] = jnp.zeros((L16,), jnp.float32)
            return carry
        lax.fori_loop(0, C, _ob, 0)

        base_row = s * ROWS_PER_TILE
        for k in range(ROWS_PER_TILE // ZR):
            pltpu.sync_copy(zero16_v, cnt_sh.at[pl.ds(base_row + k * ZR, ZR)])
        plsc.subcore_barrier()

        def _grp(g, carry):
            r0 = pl.multiple_of(wid * WCHUNKS + g * GRP, GRP)
            pltpu.sync_copy(dstr.at[pl.ds(r0, GRP)], dst_v)

            def _eb(k, carry2):
                pltpu.sync_copy(ones_v, cnt_sh.at[dst_v.at[k]], add=True)
                return carry2
            lax.fori_loop(0, GRP, _eb, 0)
            return carry
        lax.fori_loop(0, NGRP, _grp, 0)
        plsc.subcore_barrier()

        pltpu.sync_copy(cnt_sh.at[pl.ds(base_row, ROWS_PER_TILE)],
                        cnt_o.at[c, pl.ds(base_row, ROWS_PER_TILE)])

    return pl.kernel(
        body,
        out_type=jax.ShapeDtypeStruct((NC, N_ACC, D), jnp.float32),
        mesh=_sc_mesh(), scratch_types=scratch)


def _gidx_body(src_ref, typ_ref, out_ref):
    out_ref[...] = typ_ref[...] * N + src_ref[...]


_gidx_call = pl.pallas_call(
    _gidx_body,
    grid=(4,),
    in_specs=[pl.BlockSpec((NCHUNKS // 4, C), lambda i: (i, 0)),
              pl.BlockSpec((NCHUNKS // 4, C), lambda i: (i, 0))],
    out_specs=pl.BlockSpec((NCHUNKS // 4, C), lambda i: (i, 0)),
    out_shape=jax.ShapeDtypeStruct((NCHUNKS, C), jnp.int32),
)


def _upd_body(cnt_ref, p_ref, h_ref, ws_ref, out_ref):
    deg = cnt_ref[0, :, 0:1] + cnt_ref[1, :, 0:1]
    inv = 1.0 / jnp.maximum(deg, 1.0)
    agg = (p_ref[0] + p_ref[1]) * inv
    out_ref[...] = jnp.maximum(
        agg + jnp.dot(h_ref[...], ws_ref[...],
                      preferred_element_type=jnp.float32), 0.0)


_upd_call = pl.pallas_call(
    _upd_body,
    grid=(NBLK,),
    in_specs=[
        pl.BlockSpec((NC, BN, D), lambda nb: (0, nb, 0)),
        pl.BlockSpec((NC, BN, D), lambda nb: (0, nb, 0)),
        pl.BlockSpec((BN, D), lambda nb: (nb, 0)),
        pl.BlockSpec((D, D), lambda nb: (0, 0)),
    ],
    out_specs=pl.BlockSpec((BN, D), lambda nb: (nb, 0)),
    out_shape=jax.ShapeDtypeStruct((N, D), jnp.float32),
)


NG = 4096
GPW = NG // NW


def _out_body(part2, cnt, y1, idx_hbm, out_hbm,
              idx_v, p0_v, p1_v, c0_v, c1_v, y_v, sem):
    c = lax.axis_index("c")
    s = lax.axis_index("s")
    wid = c * NS + s
    base = pl.multiple_of(wid * GPW, GPW)
    pltpu.sync_copy(idx_hbm.at[pl.ds(base, GPW)], idx_v)
    cp0 = pltpu.async_copy(part2.at[0].at[idx_v], p0_v, sem)
    cp1 = pltpu.async_copy(part2.at[1].at[idx_v], p1_v, sem)
    cp2 = pltpu.async_copy(cnt.at[0].at[idx_v], c0_v, sem)
    cp3 = pltpu.async_copy(cnt.at[1].at[idx_v], c1_v, sem)
    cp4 = pltpu.async_copy(y1.at[idx_v], y_v, sem)
    cp0.wait(); cp1.wait(); cp2.wait(); cp3.wait(); cp4.wait()

    def _row(i, carry):
        for j in range(D // L16):
            sl = pl.ds(j * L16, L16)
            deg = jnp.maximum(c0_v[i, sl] + c1_v[i, sl], 1.0)
            p0_v[i, sl] = jnp.maximum(
                (p0_v[i, sl] + p1_v[i, sl]) / deg + y_v[i, sl], 0.0)
        return carry
    lax.fori_loop(0, GPW, _row, 0)
    pltpu.sync_copy(p0_v, out_hbm.at[pl.ds(base, GPW)])


@functools.lru_cache(maxsize=None)
def _make_out_kernel():
    return pl.kernel(
        _out_body,
        out_type=jax.ShapeDtypeStruct((NG, D), jnp.float32),
        mesh=_sc_mesh(),
        scratch_types=[
            pltpu.VMEM((GPW,), jnp.int32),
            pltpu.VMEM((GPW, D), jnp.float32),
            pltpu.VMEM((GPW, D), jnp.float32),
            pltpu.VMEM((GPW, D), jnp.float32),
            pltpu.VMEM((GPW, D), jnp.float32),
            pltpu.VMEM((GPW, D), jnp.float32),
            pltpu.SemaphoreType.DMA,
        ],
    )


def kernel(x, bases, comb, w_self, edge_index, edge_type, indices):
    src, dst = edge_index[0], edge_index[1]
    padn = E_PAD - E
    ar = jnp.arange(padn, dtype=jnp.int32)
    src_p = jnp.concatenate([src, (ar * 7919) % N]).reshape(NCHUNKS, C)
    typ_p = jnp.concatenate([edge_type, ar % R]).reshape(NCHUNKS, C)
    dst_p = jnp.concatenate([dst, N + ar % (N_ACC - N)]).reshape(NCHUNKS, C)

    gidx_p = _gidx_call(src_p, typ_p)
    cnt = _make_cnt_kernel()(dst_p)
    hr = _hr_call(comb[0], bases[0], x)
    part = _make_edge_kernel()(gidx_p, dst_p, hr)
    hr2, y1 = _upd_hr_call(comb[1], bases[1], cnt, part, x,
                           w_self[0], w_self[1])
    part2 = _make_edge_kernel()(gidx_p, dst_p, hr2)
    return _make_out_kernel()(part2, cnt, y1, indices)


# speedup vs baseline: 1.3516x; 1.0625x over previous
import functools

import jax
import jax.numpy as jnp
from jax import lax
from jax.experimental import pallas as pl
from jax.experimental.pallas import tpu as pltpu
from jax.experimental.pallas import tpu_sc as plsc

N = 10000
E = 320000
R = 16
NB_BASES = 8
D = 128
L16 = 16

NC, NS = 2, 16
NW = NC * NS
C = 128
WCHUNKS = (-(-E // (C * NW)) + 7) // 8 * 8
NCHUNKS = WCHUNKS * NW
E_PAD = NCHUNKS * C
N_ACC = 10240
ROWS_PER_TILE = N_ACC // NS

@functools.lru_cache(maxsize=None)
def _sc_mesh():
    return plsc.VectorSubcoreMesh(
        core_axis_name="c", subcore_axis_name="s", num_cores=NC, num_subcores=NS)


BN = 1000
NBLK = N // BN


BN_HR = 5000
NB_HR = N // BN_HR


def _hr_body(comb_ref, bases_ref, h_ref, out_ref, h_scr):
    r = pl.program_id(1)

    @pl.when(r == 0)
    def _():
        h_scr[...] = h_ref[...].astype(jnp.bfloat16)

    w = comb_ref[r, 0] * bases_ref[0]
    for b in range(1, NB_BASES):
        w += comb_ref[r, b] * bases_ref[b]
    out_ref[...] = jnp.dot(h_scr[...], w.astype(jnp.bfloat16),
                           preferred_element_type=jnp.float32)


_hr_call = pl.pallas_call(
    _hr_body,
    grid=(NB_HR, R),
    in_specs=[
        pl.BlockSpec(memory_space=pltpu.SMEM),
        pl.BlockSpec((NB_BASES, D, D), lambda nb, r: (0, 0, 0)),
        pl.BlockSpec((BN_HR, D), lambda nb, r: (nb, 0)),
    ],
    out_specs=pl.BlockSpec((BN_HR, D), lambda nb, r: (r * NB_HR + nb, 0)),
    out_shape=jax.ShapeDtypeStruct((R * N, D), jnp.float32),
    scratch_shapes=[pltpu.VMEM((BN_HR, D), jnp.bfloat16)],
)


GRP = 16
NGRP = WCHUNKS // GRP
ZR = 64
HALF = WCHUNKS // 2


@functools.lru_cache(maxsize=None)
def _make_edge_kernel():
    scratch = [
        pltpu.VMEM((HALF, C), jnp.int32),
        pltpu.VMEM((HALF, C), jnp.int32),
        pltpu.VMEM((C, D), jnp.float32),
        pltpu.VMEM((C, D), jnp.float32),
        pltpu.VMEM_SHARED((N_ACC, D), jnp.float32),
        pltpu.SemaphoreType.DMA,
        pltpu.SemaphoreType.DMA,
    ]

    def body(gidxr, dstr, hr, part_o,
             gidx_v, dst_v, rows0, rows1, acc_sh, sem0, sem1):
        c = lax.axis_index("c")
        s = lax.axis_index("s")
        wid = c * NS + s

        def _zb(i, carry):
            for j in range(D // L16):
                rows0[i, pl.ds(j * L16, L16)] = jnp.zeros((L16,), jnp.float32)
            return carry
        lax.fori_loop(0, C, _zb, 0)
        base_row = s * ROWS_PER_TILE
        for k in range(ROWS_PER_TILE // C):
            pltpu.sync_copy(rows0, acc_sh.at[pl.ds(base_row + k * C, C)])
        plsc.subcore_barrier()

        def _half(h, carry):
            r0 = pl.multiple_of(wid * WCHUNKS + h * HALF, 8)
            pltpu.sync_copy(gidxr.at[pl.ds(r0, HALF)], gidx_v)
            pltpu.sync_copy(dstr.at[pl.ds(r0, HALF)], dst_v)
            pltpu.async_copy(hr.at[gidx_v.at[0]], rows0, sem0)

            def _pair(g, carry2):
                k0 = 2 * g
                pltpu.async_copy(hr.at[gidx_v.at[k0 + 1]], rows1, sem1)
                pltpu.make_async_copy(hr.at[gidx_v.at[k0]], rows0, sem0).wait()
                pltpu.sync_copy(rows0, acc_sh.at[dst_v.at[k0]], add=True)

                @pl.when(g < HALF // 2 - 1)
                def _():
                    pltpu.async_copy(hr.at[gidx_v.at[k0 + 2]], rows0, sem0)
                pltpu.make_async_copy(hr.at[gidx_v.at[k0 + 1]], rows1,
                                      sem1).wait()
                pltpu.sync_copy(rows1, acc_sh.at[dst_v.at[k0 + 1]], add=True)
                return carry2
            lax.fori_loop(0, HALF // 2, _pair, 0)
            return carry
        lax.fori_loop(0, 2, _half, 0)
        plsc.subcore_barrier()

        pltpu.sync_copy(acc_sh.at[pl.ds(base_row, ROWS_PER_TILE)],
                        part_o.at[c, pl.ds(base_row, ROWS_PER_TILE)])

    return pl.kernel(body,
                     out_type=jax.ShapeDtypeStruct((NC, N_ACC, D), jnp.float32),
                     mesh=_sc_mesh(), scratch_types=scratch)


def _upd_hr_body(comb_ref, bases_ref, cnt_ref, p_ref, h_ref, ws_ref, ws2_ref,
                 hr_ref, y1_ref, inv_ref, h1_scr):
    r = pl.program_id(1)

    @pl.when(r == 0)
    def _():
        deg = cnt_ref[0] + cnt_ref[1]
        inv = 1.0 / jnp.maximum(deg, 1.0)
        agg = (p_ref[0] + p_ref[1]) * inv
        h1 = jnp.maximum(
            agg + jnp.dot(h_ref[...], ws_ref[...],
                          preferred_element_type=jnp.float32), 0.0)
        h1b = h1.astype(jnp.bfloat16)
        h1_scr[...] = h1b
        y1_ref[...] = jnp.dot(h1b, ws2_ref[...].astype(jnp.bfloat16),
                              preferred_element_type=jnp.float32)
        inv_ref[...] = inv * jnp.ones((1, D), jnp.float32)

    w = comb_ref[r, 0] * bases_ref[0]
    for b in range(1, NB_BASES):
        w += comb_ref[r, b] * bases_ref[b]
    hr_ref[...] = jnp.dot(h1_scr[...], w.astype(jnp.bfloat16),
                          preferred_element_type=jnp.float32)


_upd_hr_call = pl.pallas_call(
    _upd_hr_body,
    grid=(NB_HR, R),
    in_specs=[
        pl.BlockSpec(memory_space=pltpu.SMEM),
        pl.BlockSpec((NB_BASES, D, D), lambda nb, r: (0, 0, 0)),
        pl.BlockSpec((NC, BN_HR, 1), lambda nb, r: (0, nb, 0)),
        pl.BlockSpec((NC, BN_HR, D), lambda nb, r: (0, nb, 0)),
        pl.BlockSpec((BN_HR, D), lambda nb, r: (nb, 0)),
        pl.BlockSpec((D, D), lambda nb, r: (0, 0)),
        pl.BlockSpec((D, D), lambda nb, r: (0, 0)),
    ],
    out_specs=[
        pl.BlockSpec((BN_HR, D), lambda nb, r: (r * NB_HR + nb, 0)),
        pl.BlockSpec((BN_HR, D), lambda nb, r: (nb, 0)),
        pl.BlockSpec((BN_HR, D), lambda nb, r: (nb, 0)),
    ],
    out_shape=[jax.ShapeDtypeStruct((R * N, D), jnp.float32),
               jax.ShapeDtypeStruct((N, D), jnp.float32),
               jax.ShapeDtypeStruct((N, D), jnp.float32)],
    scratch_shapes=[pltpu.VMEM((BN_HR, D), jnp.bfloat16)],
)


@functools.lru_cache(maxsize=None)
def _make_cnt_kernel():
    scratch = [
        pltpu.VMEM((GRP, C), jnp.int32),
        pltpu.VMEM((C,), jnp.float32),
        pltpu.VMEM((ROWS_PER_TILE,), jnp.float32),
        pltpu.VMEM_SHARED((N_ACC,), jnp.float32),
    ]

    def body(dstr, cnt_o, dst_v, ones_v, zero_v, cnt_sh):
        c = lax.axis_index("c")
        s = lax.axis_index("s")
        wid = c * NS + s

        def _ob(i, carry):
            ones_v[pl.ds((i % (C // L16)) * L16, L16)] = jnp.ones(
                (L16,), jnp.float32)
            zero_v[pl.ds(i * L16, L16)] = jnp.zeros((L16,), jnp.float32)
            return carry
        lax.fori_loop(0, ROWS_PER_TILE // L16, _ob, 0)

        base_row = s * ROWS_PER_TILE
        pltpu.sync_copy(zero_v, cnt_sh.at[pl.ds(base_row, ROWS_PER_TILE)])
        plsc.subcore_barrier()

        def _grp(g, carry):
            r0 = pl.multiple_of(wid * WCHUNKS + g * GRP, GRP)
            pltpu.sync_copy(dstr.at[pl.ds(r0, GRP)], dst_v)

            def _eb(k, carry2):
                pltpu.sync_copy(ones_v, cnt_sh.at[dst_v.at[k]], add=True)
                return carry2
            lax.fori_loop(0, GRP, _eb, 0)
            return carry
        lax.fori_loop(0, NGRP, _grp, 0)
        plsc.subcore_barrier()

        pltpu.sync_copy(cnt_sh.at[pl.ds(base_row, ROWS_PER_TILE)],
                        cnt_o.at[c, pl.ds(base_row, ROWS_PER_TILE)])

    return pl.kernel(
        body,
        out_type=jax.ShapeDtypeStruct((NC, N_ACC), jnp.float32),
        mesh=_sc_mesh(), scratch_types=scratch)


def _gidx_body(src_ref, typ_ref, out_ref):
    out_ref[...] = typ_ref[...] * N + src_ref[...]


_gidx_call = pl.pallas_call(
    _gidx_body,
    grid=(4,),
    in_specs=[pl.BlockSpec((NCHUNKS // 4, C), lambda i: (i, 0)),
              pl.BlockSpec((NCHUNKS // 4, C), lambda i: (i, 0))],
    out_specs=pl.BlockSpec((NCHUNKS // 4, C), lambda i: (i, 0)),
    out_shape=jax.ShapeDtypeStruct((NCHUNKS, C), jnp.int32),
)


def _upd_body(cnt_ref, p_ref, h_ref, ws_ref, out_ref):
    deg = cnt_ref[0, :, 0:1] + cnt_ref[1, :, 0:1]
    inv = 1.0 / jnp.maximum(deg, 1.0)
    agg = (p_ref[0] + p_ref[1]) * inv
    out_ref[...] = jnp.maximum(
        agg + jnp.dot(h_ref[...], ws_ref[...],
                      preferred_element_type=jnp.float32), 0.0)


_upd_call = pl.pallas_call(
    _upd_body,
    grid=(NBLK,),
    in_specs=[
        pl.BlockSpec((NC, BN, D), lambda nb: (0, nb, 0)),
        pl.BlockSpec((NC, BN, D), lambda nb: (0, nb, 0)),
        pl.BlockSpec((BN, D), lambda nb: (nb, 0)),
        pl.BlockSpec((D, D), lambda nb: (0, 0)),
    ],
    out_specs=pl.BlockSpec((BN, D), lambda nb: (nb, 0)),
    out_shape=jax.ShapeDtypeStruct((N, D), jnp.float32),
)


NG = 4096
GPW = NG // NW


def _out_body(part2, inv_b, y1, idx_hbm, out_hbm,
              idx_v, p0_v, p1_v, i_v, y_v, sem):
    c = lax.axis_index("c")
    s = lax.axis_index("s")
    wid = c * NS + s
    base = pl.multiple_of(wid * GPW, GPW)
    pltpu.sync_copy(idx_hbm.at[pl.ds(base, GPW)], idx_v)
    cp0 = pltpu.async_copy(part2.at[0].at[idx_v], p0_v, sem)
    cp1 = pltpu.async_copy(part2.at[1].at[idx_v], p1_v, sem)
    cp2 = pltpu.async_copy(inv_b.at[idx_v], i_v, sem)
    cp3 = pltpu.async_copy(y1.at[idx_v], y_v, sem)
    cp0.wait(); cp1.wait(); cp2.wait(); cp3.wait()

    def _row(i, carry):
        for j in range(D // L16):
            sl = pl.ds(j * L16, L16)
            p0_v[i, sl] = jnp.maximum(
                (p0_v[i, sl] + p1_v[i, sl]) * i_v[i, sl] + y_v[i, sl], 0.0)
        return carry
    lax.fori_loop(0, GPW, _row, 0)
    pltpu.sync_copy(p0_v, out_hbm.at[pl.ds(base, GPW)])


@functools.lru_cache(maxsize=None)
def _make_out_kernel():
    return pl.kernel(
        _out_body,
        out_type=jax.ShapeDtypeStruct((NG, D), jnp.float32),
        mesh=_sc_mesh(),
        scratch_types=[
            pltpu.VMEM((GPW,), jnp.int32),
            pltpu.VMEM((GPW, D), jnp.float32),
            pltpu.VMEM((GPW, D), jnp.float32),
            pltpu.VMEM((GPW, D), jnp.float32),
            pltpu.VMEM((GPW, D), jnp.float32),
            pltpu.SemaphoreType.DMA,
        ],
    )


def kernel(x, bases, comb, w_self, edge_index, edge_type, indices):
    src, dst = edge_index[0], edge_index[1]
    padn = E_PAD - E
    ar = jnp.arange(padn, dtype=jnp.int32)
    src_p = jnp.concatenate([src, (ar * 7919) % N]).reshape(NCHUNKS, C)
    typ_p = jnp.concatenate([edge_type, ar % R]).reshape(NCHUNKS, C)
    dst_p = jnp.concatenate([dst, N + ar % (N_ACC - N)]).reshape(NCHUNKS, C)

    gidx_p = _gidx_call(src_p, typ_p)
    cnt = _make_cnt_kernel()(dst_p).reshape(NC, N_ACC, 1)
    hr = _hr_call(comb[0], bases[0], x)
    part = _make_edge_kernel()(gidx_p, dst_p, hr)
    hr2, y1, inv_b = _upd_hr_call(comb[1], bases[1], cnt, part, x,
                                  w_self[0], w_self[1])
    part2 = _make_edge_kernel()(gidx_p, dst_p, hr2)
    return _make_out_kernel()(part2, inv_b, y1, indices)


# speedup vs baseline: 1.3539x; 1.0018x over previous
import functools

import jax
import jax.numpy as jnp
from jax import lax
from jax.experimental import pallas as pl
from jax.experimental.pallas import tpu as pltpu
from jax.experimental.pallas import tpu_sc as plsc

N = 10000
E = 320000
R = 16
NB_BASES = 8
D = 128
L16 = 16

NC, NS = 2, 16
NW = NC * NS
C = 128
WCHUNKS = (-(-E // (C * NW)) + 7) // 8 * 8
NCHUNKS = WCHUNKS * NW
E_PAD = NCHUNKS * C
N_ACC = 10240
ROWS_PER_TILE = N_ACC // NS

@functools.lru_cache(maxsize=None)
def _sc_mesh():
    return plsc.VectorSubcoreMesh(
        core_axis_name="c", subcore_axis_name="s", num_cores=NC, num_subcores=NS)


BN_HR = 5000
NB_HR = N // BN_HR


def _hr_body(comb_ref, bases_ref, h_ref, out_ref, h_scr):
    r = pl.program_id(1)

    @pl.when(r == 0)
    def _():
        h_scr[...] = h_ref[...].astype(jnp.bfloat16)

    w = comb_ref[r, 0] * bases_ref[0]
    for b in range(1, NB_BASES):
        w += comb_ref[r, b] * bases_ref[b]
    out_ref[...] = jnp.dot(h_scr[...], w.astype(jnp.bfloat16),
                           preferred_element_type=jnp.float32)


_hr_call = pl.pallas_call(
    _hr_body,
    grid=(NB_HR, R),
    in_specs=[
        pl.BlockSpec(memory_space=pltpu.SMEM),
        pl.BlockSpec((NB_BASES, D, D), lambda nb, r: (0, 0, 0)),
        pl.BlockSpec((BN_HR, D), lambda nb, r: (nb, 0)),
    ],
    out_specs=pl.BlockSpec((BN_HR, D), lambda nb, r: (r * NB_HR + nb, 0)),
    out_shape=jax.ShapeDtypeStruct((R * N, D), jnp.float32),
    scratch_shapes=[pltpu.VMEM((BN_HR, D), jnp.bfloat16)],
)


GRP = 16
NGRP = WCHUNKS // GRP
ZR = 64
HALF = WCHUNKS // 2


@functools.lru_cache(maxsize=None)
def _make_edge_kernel():
    scratch = [
        pltpu.VMEM((HALF, C), jnp.int32),
        pltpu.VMEM((HALF, C), jnp.int32),
        pltpu.VMEM((C, D), jnp.float32),
        pltpu.VMEM((C, D), jnp.float32),
        pltpu.VMEM_SHARED((N_ACC, D), jnp.float32),
        pltpu.SemaphoreType.DMA,
        pltpu.SemaphoreType.DMA,
    ]

    def body(gidxr, dstr, hr, part_o,
             gidx_v, dst_v, rows0, rows1, acc_sh, sem0, sem1):
        c = lax.axis_index("c")
        s = lax.axis_index("s")
        wid = c * NS + s

        def _zb(i, carry):
            for j in range(D // L16):
                rows0[i, pl.ds(j * L16, L16)] = jnp.zeros((L16,), jnp.float32)
            return carry
        lax.fori_loop(0, C, _zb, 0)
        base_row = s * ROWS_PER_TILE
        for k in range(ROWS_PER_TILE // C):
            pltpu.sync_copy(rows0, acc_sh.at[pl.ds(base_row + k * C, C)])
        plsc.subcore_barrier()

        def _half(h, carry):
            r0 = pl.multiple_of(wid * WCHUNKS + h * HALF, 8)
            pltpu.sync_copy(gidxr.at[pl.ds(r0, HALF)], gidx_v)
            pltpu.sync_copy(dstr.at[pl.ds(r0, HALF)], dst_v)
            pltpu.async_copy(hr.at[gidx_v.at[0]], rows0, sem0)

            def _pair(g, carry2):
                k0 = 2 * g
                pltpu.async_copy(hr.at[gidx_v.at[k0 + 1]], rows1, sem1)
                pltpu.make_async_copy(hr.at[gidx_v.at[k0]], rows0, sem0).wait()
                pltpu.sync_copy(rows0, acc_sh.at[dst_v.at[k0]], add=True)

                @pl.when(g < HALF // 2 - 1)
                def _():
                    pltpu.async_copy(hr.at[gidx_v.at[k0 + 2]], rows0, sem0)
                pltpu.make_async_copy(hr.at[gidx_v.at[k0 + 1]], rows1,
                                      sem1).wait()
                pltpu.sync_copy(rows1, acc_sh.at[dst_v.at[k0 + 1]], add=True)
                return carry2
            lax.fori_loop(0, HALF // 2, _pair, 0)
            return carry
        lax.fori_loop(0, 2, _half, 0)
        plsc.subcore_barrier()

        pltpu.sync_copy(acc_sh.at[pl.ds(base_row, ROWS_PER_TILE)],
                        part_o.at[c, pl.ds(base_row, ROWS_PER_TILE)])

    return pl.kernel(body,
                     out_type=jax.ShapeDtypeStruct((NC, N_ACC, D), jnp.float32),
                     mesh=_sc_mesh(), scratch_types=scratch)


def _upd_hr_body(comb_ref, bases_ref, cnt_ref, p_ref, h_ref, ws_ref, ws2_ref,
                 hr_ref, y1_ref, inv_ref, h1_scr):
    r = pl.program_id(1)

    @pl.when(r == 0)
    def _():
        deg = cnt_ref[0] + cnt_ref[1]
        inv = 1.0 / jnp.maximum(deg, 1.0)
        agg = (p_ref[0] + p_ref[1]) * inv
        h1 = jnp.maximum(
            agg + jnp.dot(h_ref[...], ws_ref[...],
                          preferred_element_type=jnp.float32), 0.0)
        h1b = h1.astype(jnp.bfloat16)
        h1_scr[...] = h1b
        y1_ref[...] = jnp.dot(h1b, ws2_ref[...].astype(jnp.bfloat16),
                              preferred_element_type=jnp.float32)
        inv_ref[...] = inv * jnp.ones((1, D), jnp.float32)

    w = comb_ref[r, 0] * bases_ref[0]
    for b in range(1, NB_BASES):
        w += comb_ref[r, b] * bases_ref[b]
    hr_ref[...] = jnp.dot(h1_scr[...], w.astype(jnp.bfloat16),
                          preferred_element_type=jnp.float32)


_upd_hr_call = pl.pallas_call(
    _upd_hr_body,
    grid=(NB_HR, R),
    in_specs=[
        pl.BlockSpec(memory_space=pltpu.SMEM),
        pl.BlockSpec((NB_BASES, D, D), lambda nb, r: (0, 0, 0)),
        pl.BlockSpec((NC, BN_HR, 1), lambda nb, r: (0, nb, 0)),
        pl.BlockSpec((NC, BN_HR, D), lambda nb, r: (0, nb, 0)),
        pl.BlockSpec((BN_HR, D), lambda nb, r: (nb, 0)),
        pl.BlockSpec((D, D), lambda nb, r: (0, 0)),
        pl.BlockSpec((D, D), lambda nb, r: (0, 0)),
    ],
    out_specs=[
        pl.BlockSpec((BN_HR, D), lambda nb, r: (r * NB_HR + nb, 0)),
        pl.BlockSpec((BN_HR, D), lambda nb, r: (nb, 0)),
        pl.BlockSpec((BN_HR, D), lambda nb, r: (nb, 0)),
    ],
    out_shape=[jax.ShapeDtypeStruct((R * N, D), jnp.float32),
               jax.ShapeDtypeStruct((N, D), jnp.float32),
               jax.ShapeDtypeStruct((N, D), jnp.float32)],
    scratch_shapes=[pltpu.VMEM((BN_HR, D), jnp.bfloat16)],
)


@functools.lru_cache(maxsize=None)
def _make_cnt_kernel():
    scratch = [
        pltpu.VMEM((GRP, C), jnp.int32),
        pltpu.VMEM((C,), jnp.float32),
        pltpu.VMEM((ROWS_PER_TILE,), jnp.float32),
        pltpu.VMEM_SHARED((N_ACC,), jnp.float32),
    ]

    def body(dstr, cnt_o, dst_v, ones_v, zero_v, cnt_sh):
        c = lax.axis_index("c")
        s = lax.axis_index("s")
        wid = c * NS + s

        def _ob(i, carry):
            ones_v[pl.ds((i % (C // L16)) * L16, L16)] = jnp.ones(
                (L16,), jnp.float32)
            zero_v[pl.ds(i * L16, L16)] = jnp.zeros((L16,), jnp.float32)
            return carry
        lax.fori_loop(0, ROWS_PER_TILE // L16, _ob, 0)

        base_row = s * ROWS_PER_TILE
        pltpu.sync_copy(zero_v, cnt_sh.at[pl.ds(base_row, ROWS_PER_TILE)])
        plsc.subcore_barrier()

        def _grp(g, carry):
            r0 = pl.multiple_of(wid * WCHUNKS + g * GRP, GRP)
            pltpu.sync_copy(dstr.at[pl.ds(r0, GRP)], dst_v)

            def _eb(k, carry2):
                pltpu.sync_copy(ones_v, cnt_sh.at[dst_v.at[k]], add=True)
                return carry2
            lax.fori_loop(0, GRP, _eb, 0)
            return carry
        lax.fori_loop(0, NGRP, _grp, 0)
        plsc.subcore_barrier()

        pltpu.sync_copy(cnt_sh.at[pl.ds(base_row, ROWS_PER_TILE)],
                        cnt_o.at[c, pl.ds(base_row, ROWS_PER_TILE)])

    return pl.kernel(
        body,
        out_type=jax.ShapeDtypeStruct((NC, N_ACC), jnp.float32),
        mesh=_sc_mesh(), scratch_types=scratch)


def _gidx_body(src_ref, typ_ref, out_ref):
    out_ref[...] = typ_ref[...] * N + src_ref[...]


_gidx_call = pl.pallas_call(
    _gidx_body,
    grid=(4,),
    in_specs=[pl.BlockSpec((NCHUNKS // 4, C), lambda i: (i, 0)),
              pl.BlockSpec((NCHUNKS // 4, C), lambda i: (i, 0))],
    out_specs=pl.BlockSpec((NCHUNKS // 4, C), lambda i: (i, 0)),
    out_shape=jax.ShapeDtypeStruct((NCHUNKS, C), jnp.int32),
)


NG = 4096
GPW = NG // NW


def _out_body(part2, inv_b, y1, idx_hbm, out_hbm,
              idx_v, p0_v, p1_v, i_v, y_v, sem):
    c = lax.axis_index("c")
    s = lax.axis_index("s")
    wid = c * NS + s
    base = pl.multiple_of(wid * GPW, GPW)
    pltpu.sync_copy(idx_hbm.at[pl.ds(base, GPW)], idx_v)
    cp0 = pltpu.async_copy(part2.at[0].at[idx_v], p0_v, sem)
    cp1 = pltpu.async_copy(part2.at[1].at[idx_v], p1_v, sem)
    cp2 = pltpu.async_copy(inv_b.at[idx_v], i_v, sem)
    cp3 = pltpu.async_copy(y1.at[idx_v], y_v, sem)
    cp0.wait(); cp1.wait(); cp2.wait(); cp3.wait()

    def _row(i, carry):
        for j in range(D // L16):
            sl = pl.ds(j * L16, L16)
            p0_v[i, sl] = jnp.maximum(
                (p0_v[i, sl] + p1_v[i, sl]) * i_v[i, sl] + y_v[i, sl], 0.0)
        return carry
    lax.fori_loop(0, GPW, _row, 0)
    pltpu.sync_copy(p0_v, out_hbm.at[pl.ds(base, GPW)])


@functools.lru_cache(maxsize=None)
def _make_out_kernel():
    return pl.kernel(
        _out_body,
        out_type=jax.ShapeDtypeStruct((NG, D), jnp.float32),
        mesh=_sc_mesh(),
        scratch_types=[
            pltpu.VMEM((GPW,), jnp.int32),
            pltpu.VMEM((GPW, D), jnp.float32),
            pltpu.VMEM((GPW, D), jnp.float32),
            pltpu.VMEM((GPW, D), jnp.float32),
            pltpu.VMEM((GPW, D), jnp.float32),
            pltpu.SemaphoreType.DMA,
        ],
    )


def kernel(x, bases, comb, w_self, edge_index, edge_type, indices):
    src, dst = edge_index[0], edge_index[1]
    padn = E_PAD - E
    ar = jnp.arange(padn, dtype=jnp.int32)
    src_p = jnp.concatenate([src, (ar * 7919) % N]).reshape(NCHUNKS, C)
    typ_p = jnp.concatenate([edge_type, ar % R]).reshape(NCHUNKS, C)
    dst_p = jnp.concatenate([dst, N + ar % (N_ACC - N)]).reshape(NCHUNKS, C)

    gidx_p = _gidx_call(src_p, typ_p)
    cnt = _make_cnt_kernel()(dst_p).reshape(NC, N_ACC, 1)
    hr = _hr_call(comb[0], bases[0], x)
    part = _make_edge_kernel()(gidx_p, dst_p, hr)
    hr2, y1, inv_b = _upd_hr_call(comb[1], bases[1], cnt, part, x,
                                  w_self[0], w_self[1])
    part2 = _make_edge_kernel()(gidx_p, dst_p, hr2)
    return _make_out_kernel()(part2, inv_b, y1, indices)
